# baseline (speedup 1.0000x reference)
import functools

import jax
import jax.numpy as jnp
from jax import lax
from jax.experimental import pallas as pl
from jax.experimental.pallas import tpu as pltpu
from jax.experimental.pallas import tpu_sc as plsc

N_FIELDS = 26
K = 16
BATCH = 4096
DIN = N_FIELDS * K

NC, NS = 2, 16
NW = NC * NS
BPW = BATCH // NW
RPW = BPW * N_FIELDS
CHUNK = 128
NCHUNK = RPW // CHUNK


def _sc_gather(xt2d, emb_v, w1):
    mesh = plsc.VectorSubcoreMesh(core_axis_name="c", subcore_axis_name="s")

    @functools.partial(
        pl.kernel,
        mesh=mesh,
        out_type=[
            jax.ShapeDtypeStruct((BATCH * N_FIELDS, K), jnp.float32),
            jax.ShapeDtypeStruct((BATCH * N_FIELDS,), jnp.float32),
        ],
        scratch_types=[
            pltpu.VMEM((NCHUNK, CHUNK), jnp.int32),
            pltpu.VMEM((RPW, K), jnp.float32),
            pltpu.VMEM((RPW,), jnp.float32),
            pltpu.SemaphoreType.DMA,
            pltpu.SemaphoreType.DMA,
        ],
        compiler_params=pltpu.CompilerParams(use_tc_tiling_on_sc=False),
    )
    def k(xt_hbm, emb_hbm, w1_hbm, deep_out, w1g_out,
          idx_v, rows_v, w1r_v, sem_e, sem_w):
        wid = lax.axis_index("s") * NC + lax.axis_index("c")
        pltpu.sync_copy(xt_hbm.at[wid], idx_v)
        copies = []
        for c in range(NCHUNK):
            copies.append(pltpu.async_copy(
                emb_hbm.at[idx_v.at[c]],
                rows_v.at[pl.ds(c * CHUNK, CHUNK)], sem_e))
            copies.append(pltpu.async_copy(
                w1_hbm.at[idx_v.at[c]],
                w1r_v.at[pl.ds(c * CHUNK, CHUNK)], sem_w))
        for cp in copies:
            cp.wait()
        base = wid * RPW
        pltpu.sync_copy(rows_v, deep_out.at[pl.ds(base, RPW)])
        pltpu.sync_copy(w1r_v, w1g_out.at[pl.ds(base, RPW)])

    return k(xt2d, emb_v, w1)


BM = 512


def _tc_body(deep_ref, w1g_ref, w0_ref, W1_ref, b1_ref, W2_ref, b2_ref,
             Wout_ref, S_ref, out_ref):
    d = deep_ref[...]
    sumV = jnp.dot(d, S_ref[...], preferred_element_type=jnp.float32)
    s2 = jnp.sum(sumV * sumV, axis=1, keepdims=True)
    sq = jnp.sum(d * d, axis=1, keepdims=True)
    fm2 = (s2 - sq) * 0.5
    fm1 = jnp.sum(w1g_ref[...], axis=1, keepdims=True)
    h = jnp.maximum(
        jnp.dot(d, W1_ref[...], preferred_element_type=jnp.float32)
        + b1_ref[...], 0.0)
    h = jnp.maximum(
        jnp.dot(h, W2_ref[...], preferred_element_type=jnp.float32)
        + b2_ref[...], 0.0)
    logit = (jnp.dot(h, Wout_ref[...], preferred_element_type=jnp.float32)
             + w0_ref[...] + fm1 + fm2)
    out_ref[...] = 1.0 / (1.0 + jnp.exp(-logit))


def _tc_mlp(deep, w1g, w0, W1, b1, W2, b2, Wout, S):
    h1 = W1.shape[1]
    h2 = W2.shape[1]
    return pl.pallas_call(
        _tc_body,
        grid=(BATCH // BM,),
        in_specs=[
            pl.BlockSpec((BM, DIN), lambda i: (i, 0)),
            pl.BlockSpec((BM, N_FIELDS), lambda i: (i, 0)),
            pl.BlockSpec((1, 1), lambda i: (0, 0)),
            pl.BlockSpec((DIN, h1), lambda i: (0, 0)),
            pl.BlockSpec((1, h1), lambda i: (0, 0)),
            pl.BlockSpec((h1, h2), lambda i: (0, 0)),
            pl.BlockSpec((1, h2), lambda i: (0, 0)),
            pl.BlockSpec((h2, 1), lambda i: (0, 0)),
            pl.BlockSpec((DIN, K), lambda i: (0, 0)),
        ],
        out_specs=pl.BlockSpec((BM, 1), lambda i: (i, 0)),
        out_shape=jax.ShapeDtypeStruct((BATCH, 1), jnp.float32),
    )(deep, w1g, w0, W1, b1, W2, b2, Wout, S)


def kernel(x, emb_v, w0, w1, W_h1, b_h1, W_h2, b_h2, W_out):
    xt2d = x.T.reshape(NW, NCHUNK, CHUNK)
    deep2d, w1r = _sc_gather(xt2d, emb_v, w1.reshape(-1))
    deep = deep2d.reshape(BATCH, DIN)
    w1g = w1r.reshape(BATCH, N_FIELDS)
    S = jnp.tile(jnp.eye(K, dtype=jnp.float32), (N_FIELDS, 1))
    return _tc_mlp(deep, w1g, jnp.reshape(w0, (1, 1)), W_h1,
                   b_h1.reshape(1, -1), W_h2, b_h2.reshape(1, -1), W_out, S)


# speedup vs baseline: 1.0169x; 1.0169x over previous
import functools

import jax
import jax.numpy as jnp
from jax import lax
from jax.experimental import pallas as pl
from jax.experimental.pallas import tpu as pltpu
from jax.experimental.pallas import tpu_sc as plsc

N_FIELDS = 26
K = 16
BATCH = 4096
FPAD = 32
DPAD = FPAD * K
NJ = DPAD // 128
HASH = 1000000

NC, NS = 2, 16
NW = NC * NS
BPW = BATCH // NW
NSTREAM = NJ * (BPW // K)
RPT = BPW * FPAD


def _sc_gather(xq3, xp3, emb_v, w1):
    mesh = plsc.VectorSubcoreMesh(core_axis_name="c", subcore_axis_name="s")

    @functools.partial(
        pl.kernel,
        mesh=mesh,
        out_type=[jax.ShapeDtypeStruct((BATCH * 8, K), jnp.float32)
                  for _ in range(NJ)]
        + [jax.ShapeDtypeStruct((BATCH, FPAD), jnp.float32)],
        scratch_types=[
            pltpu.VMEM((NSTREAM, 128), jnp.int32),
            pltpu.VMEM((BPW, FPAD), jnp.int32),
            pltpu.VMEM((RPT, K), jnp.float32),
            pltpu.VMEM((BPW, FPAD), jnp.float32),
            pltpu.SemaphoreType.DMA,
            pltpu.SemaphoreType.DMA,
        ],
        compiler_params=pltpu.CompilerParams(use_tc_tiling_on_sc=False),
    )
    def k(xq_hbm, xp_hbm, emb_hbm, w1_hbm, d0_out, d1_out, d2_out, d3_out,
          w1g_out, xq_v, xp_v, stag_v, w1r_v, sem_e, sem_w):
        wid = lax.axis_index("s") * NC + lax.axis_index("c")
        pltpu.sync_copy(xq_hbm.at[wid], xq_v)
        pltpu.sync_copy(xp_hbm.at[wid], xp_v)

        def fire_e(s, _):
            pltpu.async_copy(emb_hbm.at[xq_v.at[s]],
                             stag_v.at[pl.ds(s * 128, 128)], sem_e)
            return 0

        lax.fori_loop(0, NSTREAM, fire_e, 0)

        def fire_w(b, _):
            pltpu.async_copy(w1_hbm.at[xp_v.at[b]], w1r_v.at[b], sem_w)
            return 0

        lax.fori_loop(0, BPW, fire_w, 0)
        pltpu.make_async_copy(
            d0_out.at[pl.ds(0, RPT)], stag_v, sem_e).wait()
        pltpu.make_async_copy(
            w1g_out.at[pl.ds(0, BPW)], w1r_v, sem_w).wait()
        qb = wid * (8 * BPW)
        pltpu.sync_copy(stag_v.at[pl.ds(0, 1024)], d0_out.at[pl.ds(qb, 1024)])
        pltpu.sync_copy(stag_v.at[pl.ds(1024, 1024)], d1_out.at[pl.ds(qb, 1024)])
        pltpu.sync_copy(stag_v.at[pl.ds(2048, 1024)], d2_out.at[pl.ds(qb, 1024)])
        pltpu.sync_copy(stag_v.at[pl.ds(3072, 1024)], d3_out.at[pl.ds(qb, 1024)])
        pltpu.sync_copy(w1r_v, w1g_out.at[pl.ds(wid * BPW, BPW)])

    return k(xq3, xp3, emb_v, w1)


BM = 512


def _tc_body(d0_ref, d1_ref, d2_ref, d3_ref, w1g_ref, w0_ref, W1_ref, b1_ref,
             W2_ref, b2_ref, Wout_ref, S_ref, out_ref):
    lane = lax.broadcasted_iota(jnp.int32, (1, 128), 1)
    d3m = jnp.where(lane < 32, d3_ref[...], 0.0)
    d = jnp.concatenate(
        [d0_ref[...], d1_ref[...], d2_ref[...], d3m], axis=1
    )
    sumV = jnp.dot(d, S_ref[...], preferred_element_type=jnp.float32)
    s2 = jnp.sum(sumV * sumV, axis=1, keepdims=True)
    sq = jnp.sum(d * d, axis=1, keepdims=True)
    fm2 = (s2 - sq) * 0.5
    lane32 = lax.broadcasted_iota(jnp.int32, (1, FPAD), 1)
    w1m = jnp.where(lane32 < N_FIELDS, w1g_ref[...], 0.0)
    fm1 = jnp.sum(w1m, axis=1, keepdims=True)
    h = jnp.maximum(
        jnp.dot(d, W1_ref[...], preferred_element_type=jnp.float32)
        + b1_ref[...], 0.0)
    h = jnp.maximum(
        jnp.dot(h, W2_ref[...], preferred_element_type=jnp.float32)
        + b2_ref[...], 0.0)
    logit = (jnp.dot(h, Wout_ref[...], preferred_element_type=jnp.float32)
             + w0_ref[...] + fm1 + fm2)
    out_ref[...] = 1.0 / (1.0 + jnp.exp(-logit))


def _tc_mlp(d0, d1, d2, d3, w1g, w0, W1p, b1, W2, b2, Wout, S):
    h1 = W1p.shape[1]
    h2 = W2.shape[1]
    dspec = pl.BlockSpec((BM, 128), lambda i: (i, 0))
    return pl.pallas_call(
        _tc_body,
        grid=(BATCH // BM,),
        in_specs=[
            dspec, dspec, dspec, dspec,
            pl.BlockSpec((BM, FPAD), lambda i: (i, 0)),
            pl.BlockSpec((1, 1), lambda i: (0, 0)),
            pl.BlockSpec((DPAD, h1), lambda i: (0, 0)),
            pl.BlockSpec((1, h1), lambda i: (0, 0)),
            pl.BlockSpec((h1, h2), lambda i: (0, 0)),
            pl.BlockSpec((1, h2), lambda i: (0, 0)),
            pl.BlockSpec((h2, 1), lambda i: (0, 0)),
            pl.BlockSpec((DPAD, K), lambda i: (0, 0)),
        ],
        out_specs=pl.BlockSpec((BM, 1), lambda i: (i, 0)),
        out_shape=jax.ShapeDtypeStruct((BATCH, 1), jnp.float32),
    )(d0, d1, d2, d3, w1g, w0, W1p, b1, W2, b2, Wout, S)


def kernel(x, emb_v, w0, w1, W_h1, b_h1, W_h2, b_h2, W_out):
    pad = (jax.lax.broadcasted_iota(jnp.int32, (FPAD - N_FIELDS, BATCH), 0)
           + jax.lax.broadcasted_iota(jnp.int32, (FPAD - N_FIELDS, BATCH), 1)
           * 13) % HASH
    xpad = jnp.concatenate([x, pad], axis=0)
    xq3 = (xpad.reshape(NJ, 8, NW, 8, K)
           .transpose(2, 0, 3, 4, 1)
           .reshape(NW, NSTREAM, 128))
    xp3 = xpad.T.reshape(NW, BPW, FPAD)
    d0, d1, d2, d3, w1g = _sc_gather(xq3, xp3, emb_v, w1.reshape(-1))
    d0 = d0.reshape(BATCH, 128)
    d1 = d1.reshape(BATCH, 128)
    d2 = d2.reshape(BATCH, 128)
    d3 = d3.reshape(BATCH, 128)
    W1f = W_h1.reshape(N_FIELDS, K, -1)
    W1p = jnp.zeros((FPAD, K, W_h1.shape[1]), jnp.float32).at[:N_FIELDS].set(
        W1f).reshape(DPAD, -1)
    S = jnp.zeros((FPAD, K, K), jnp.float32).at[:N_FIELDS].set(
        jnp.broadcast_to(jnp.eye(K, dtype=jnp.float32), (N_FIELDS, K, K))
    ).reshape(DPAD, K)
    return _tc_mlp(d0, d1, d2, d3, w1g, jnp.reshape(w0, (1, 1)), W1p,
                   b_h1.reshape(1, -1), W_h2, b_h2.reshape(1, -1), W_out, S)
